# baseline (speedup 1.0000x reference)
import functools

import jax
import jax.numpy as jnp
from jax import lax
from jax.experimental import pallas as pl
from jax.experimental.pallas import tpu as pltpu
from jax.experimental.pallas import tpu_sc as plsc

EPS = 1e-12

_NC = 2
_NS = 16
_NW = _NC * _NS
_NCHUNK = 2


def _sc_gather(input_ids, table):
    b, s = input_ids.shape
    n = b * s
    d = table.shape[1]
    per_w = n // _NW
    ck = per_w // _NCHUNK
    segs_per_row = s // per_w
    mesh = plsc.VectorSubcoreMesh(core_axis_name="c", subcore_axis_name="s")

    @functools.partial(
        pl.kernel,
        mesh=mesh,
        out_type=jax.ShapeDtypeStruct((n, d), jnp.float32),
        scratch_types=[
            pltpu.VMEM((per_w,), jnp.int32),
            pltpu.VMEM((_NCHUNK, ck, d), jnp.float32),
            pltpu.SemaphoreType.DMA((_NCHUNK,)),
            pltpu.SemaphoreType.DMA,
        ],
    )
    def gather_k(idx_hbm, table_hbm, out_hbm, idx_v, rows_v, gsems, wsem):
        wid = lax.axis_index("s") * _NC + lax.axis_index("c")
        row = wid // segs_per_row
        col = (wid % segs_per_row) * per_w
        base = wid * per_w
        pltpu.sync_copy(idx_hbm.at[row, pl.ds(col, per_w)], idx_v)
        gathers = []
        for k in range(_NCHUNK):
            gathers.append(pltpu.async_copy(
                table_hbm.at[idx_v.at[pl.ds(k * ck, ck)]],
                rows_v.at[k], gsems.at[k]))
        for k in range(_NCHUNK):
            gathers[k].wait()
            pltpu.async_copy(rows_v.at[k], out_hbm.at[pl.ds(base + k * ck, ck)], wsem)
        for k in range(_NCHUNK):
            pltpu.make_async_copy(rows_v.at[k], out_hbm.at[pl.ds(base + k * ck, ck)], wsem).wait()

    return gather_k(input_ids, table)


def _tc_layernorm(gathered, pos_table, type_embeddings, gamma, beta, s):
    n, d = gathered.shape
    tb = s
    sp = 1

    def body(g_ref, pos_ref, typ_ref, gam_ref, bet_ref, o_ref):
        x = g_ref[...].reshape(sp, s, d) + pos_ref[...][None] + typ_ref[0:1, :][None]
        mean = jnp.mean(x, axis=-1, keepdims=True)
        xc = x - mean
        var = jnp.mean(xc * xc, axis=-1, keepdims=True)
        y = (xc * lax.rsqrt(var + EPS)) * gam_ref[...] + bet_ref[...]
        o_ref[...] = y.reshape(tb, d)

    typ = type_embeddings
    return pl.pallas_call(
        body,
        grid=(n // tb,),
        in_specs=[
            pl.BlockSpec((tb, d), lambda i: (i, 0)),
            pl.BlockSpec((s, d), lambda i: (0, 0)),
            pl.BlockSpec(typ.shape, lambda i: (0, 0)),
            pl.BlockSpec((d,), lambda i: (0,)),
            pl.BlockSpec((d,), lambda i: (0,)),
        ],
        out_specs=pl.BlockSpec((tb, d), lambda i: (i, 0)),
        out_shape=jax.ShapeDtypeStruct((n, d), jnp.float32),
    )(gathered, pos_table, typ, gamma, beta)


def kernel(input_ids, word_embeddings, position_embeddings, type_embeddings, gamma, beta):
    b, s = input_ids.shape
    d = word_embeddings.shape[1]

    gathered = _sc_gather(input_ids, word_embeddings)
    out = _tc_layernorm(gathered, position_embeddings, type_embeddings, gamma, beta, s)
    return out.reshape(b, s, d)


# speedup vs baseline: 1.0262x; 1.0262x over previous
import functools

import jax
import jax.numpy as jnp
from jax import lax
from jax.experimental import pallas as pl
from jax.experimental.pallas import tpu as pltpu
from jax.experimental.pallas import tpu_sc as plsc

EPS = 1e-12

_NC = 2
_NS = 16
_NW = _NC * _NS
_NCHUNK = 2


def _sc_gather(input_ids, table):
    b, s = input_ids.shape
    n = b * s
    d = table.shape[1]
    per_w = n // _NW
    ck = per_w // _NCHUNK
    segs_per_row = s // per_w
    mesh = plsc.VectorSubcoreMesh(core_axis_name="c", subcore_axis_name="s")

    @functools.partial(
        pl.kernel,
        mesh=mesh,
        out_type=jax.ShapeDtypeStruct((n, d), jnp.float32),
        scratch_types=[
            pltpu.VMEM((per_w,), jnp.int32),
            pltpu.VMEM((_NCHUNK, ck, d), jnp.float32),
            pltpu.SemaphoreType.DMA((_NCHUNK,)),
            pltpu.SemaphoreType.DMA,
        ],
    )
    def gather_k(idx_hbm, table_hbm, out_hbm, idx_v, rows_v, gsems, wsem):
        wid = lax.axis_index("s") * _NC + lax.axis_index("c")
        row = wid // segs_per_row
        col = (wid % segs_per_row) * per_w
        base = wid * per_w
        pltpu.sync_copy(idx_hbm.at[row, pl.ds(col, per_w)], idx_v)
        gathers = []
        for k in range(_NCHUNK):
            gathers.append(pltpu.async_copy(
                table_hbm.at[idx_v.at[pl.ds(k * ck, ck)]],
                rows_v.at[k], gsems.at[k]))
        for k in range(_NCHUNK):
            gathers[k].wait()
            pltpu.async_copy(rows_v.at[k], out_hbm.at[pl.ds(base + k * ck, ck)], wsem)
        for k in range(_NCHUNK):
            pltpu.make_async_copy(rows_v.at[k], out_hbm.at[pl.ds(base + k * ck, ck)], wsem).wait()

    return gather_k(input_ids, table)


def _tc_layernorm(gathered, pos_table, type_embeddings, gamma, beta, s):
    n, d = gathered.shape
    tb = 2 * s
    sp = tb // s

    def body(g_ref, pos_ref, typ_ref, gam_ref, bet_ref, o_ref):
        x = g_ref[...].reshape(sp, s, d) + pos_ref[...][None] + typ_ref[0:1, :][None]
        mean = jnp.mean(x, axis=-1, keepdims=True)
        xc = x - mean
        var = jnp.mean(xc * xc, axis=-1, keepdims=True)
        y = (xc * lax.rsqrt(var + EPS)) * gam_ref[...] + bet_ref[...]
        o_ref[...] = y.reshape(tb, d)

    typ = type_embeddings
    return pl.pallas_call(
        body,
        grid=(n // tb,),
        in_specs=[
            pl.BlockSpec((tb, d), lambda i: (i, 0)),
            pl.BlockSpec((s, d), lambda i: (0, 0)),
            pl.BlockSpec(typ.shape, lambda i: (0, 0)),
            pl.BlockSpec((d,), lambda i: (0,)),
            pl.BlockSpec((d,), lambda i: (0,)),
        ],
        out_specs=pl.BlockSpec((tb, d), lambda i: (i, 0)),
        out_shape=jax.ShapeDtypeStruct((n, d), jnp.float32),
    )(gathered, pos_table, typ, gamma, beta)


def kernel(input_ids, word_embeddings, position_embeddings, type_embeddings, gamma, beta):
    b, s = input_ids.shape
    d = word_embeddings.shape[1]

    gathered = _sc_gather(input_ids, word_embeddings)
    out = _tc_layernorm(gathered, position_embeddings, type_embeddings, gamma, beta, s)
    return out.reshape(b, s, d)


# speedup vs baseline: 1.0273x; 1.0011x over previous
import functools

import jax
import jax.numpy as jnp
from jax import lax
from jax.experimental import pallas as pl
from jax.experimental.pallas import tpu as pltpu
from jax.experimental.pallas import tpu_sc as plsc

EPS = 1e-12

_NC = 2
_NS = 16
_NW = _NC * _NS
_NCHUNK = 2


def _sc_gather(input_ids, table):
    b, s = input_ids.shape
    n = b * s
    d = table.shape[1]
    per_w = n // _NW
    ck = per_w // _NCHUNK
    segs_per_row = s // per_w
    mesh = plsc.VectorSubcoreMesh(core_axis_name="c", subcore_axis_name="s")

    @functools.partial(
        pl.kernel,
        mesh=mesh,
        out_type=jax.ShapeDtypeStruct((n, d), jnp.float32),
        scratch_types=[
            pltpu.VMEM((_NCHUNK, ck), jnp.int32),
            pltpu.VMEM((_NCHUNK, ck, d), jnp.float32),
            pltpu.SemaphoreType.DMA((_NCHUNK,)),
            pltpu.SemaphoreType.DMA,
        ],
    )
    def gather_k(idx_hbm, table_hbm, out_hbm, idx_v, rows_v, gsems, wsem):
        wid = lax.axis_index("s") * _NC + lax.axis_index("c")
        row = wid // segs_per_row
        col = (wid % segs_per_row) * per_w
        base = wid * per_w
        gathers = []
        for k in range(_NCHUNK):
            pltpu.sync_copy(idx_hbm.at[row, pl.ds(col + k * ck, ck)], idx_v.at[k])
            gathers.append(pltpu.async_copy(
                table_hbm.at[idx_v.at[k]], rows_v.at[k], gsems.at[k]))
        for k in range(_NCHUNK):
            gathers[k].wait()
            pltpu.async_copy(rows_v.at[k], out_hbm.at[pl.ds(base + k * ck, ck)], wsem)
        for k in range(_NCHUNK):
            pltpu.make_async_copy(rows_v.at[k], out_hbm.at[pl.ds(base + k * ck, ck)], wsem).wait()

    return gather_k(input_ids, table)


def _tc_layernorm(gathered, pos_table, type_embeddings, gamma, beta, s):
    n, d = gathered.shape
    tb = 2 * s
    sp = tb // s

    def body(g_ref, pos_ref, typ_ref, gam_ref, bet_ref, o_ref):
        x = g_ref[...].reshape(sp, s, d) + pos_ref[...][None] + typ_ref[0:1, :][None]
        mean = jnp.mean(x, axis=-1, keepdims=True)
        xc = x - mean
        var = jnp.mean(xc * xc, axis=-1, keepdims=True)
        y = (xc * lax.rsqrt(var + EPS)) * gam_ref[...] + bet_ref[...]
        o_ref[...] = y.reshape(tb, d)

    typ = type_embeddings
    return pl.pallas_call(
        body,
        grid=(n // tb,),
        in_specs=[
            pl.BlockSpec((tb, d), lambda i: (i, 0)),
            pl.BlockSpec((s, d), lambda i: (0, 0)),
            pl.BlockSpec(typ.shape, lambda i: (0, 0)),
            pl.BlockSpec((d,), lambda i: (0,)),
            pl.BlockSpec((d,), lambda i: (0,)),
        ],
        out_specs=pl.BlockSpec((tb, d), lambda i: (i, 0)),
        out_shape=jax.ShapeDtypeStruct((n, d), jnp.float32),
    )(gathered, pos_table, typ, gamma, beta)


def kernel(input_ids, word_embeddings, position_embeddings, type_embeddings, gamma, beta):
    b, s = input_ids.shape
    d = word_embeddings.shape[1]

    gathered = _sc_gather(input_ids, word_embeddings)
    out = _tc_layernorm(gathered, position_embeddings, type_embeddings, gamma, beta, s)
    return out.reshape(b, s, d)
